# baseline (speedup 1.0000x reference)
import functools

import jax
import jax.numpy as jnp
from jax import lax
from jax.experimental import pallas as pl
from jax.experimental.pallas import tpu as pltpu
from jax.experimental.pallas import tpu_sc as plsc

N_NODES = 10000
M = 10240
E_EDGES = 160000
BM = 256
GRID_M = M // BM
PW = 128

F32 = jnp.float32
BF16 = jnp.bfloat16


def _bdot(a, b):
    return jnp.dot(a.astype(BF16), b.astype(BF16), preferred_element_type=F32)


def _mlp_body(nf, emb, w1a, w1b, b1, w2, b2, o1, o2):
    m = pl.program_id(0)
    h = _bdot(nf[...], w1a[...]) + _bdot(emb[...][:, :64], w1b[...]) + b1[...]
    h = jnp.maximum(h, 0.0)
    x = _bdot(h, w2[...]) + b2[...]
    rows = m * BM + lax.broadcasted_iota(jnp.int32, (BM, 1), 0)
    maskf = (rows < N_NODES).astype(F32)
    x = x * maskf
    o1[...] = x[:, :PW]
    o2[...] = x[:, PW:2 * PW]


def _run_mlp(nf_pad, emb_g, w1a, w1b, b1, w2, b2):
    din_a = nf_pad.shape[1]
    exp, gin = w1a.shape[1], w2.shape[1]
    return pl.pallas_call(
        _mlp_body,
        grid=(GRID_M,),
        in_specs=[
            pl.BlockSpec((BM, din_a), lambda m: (m, 0)),
            pl.BlockSpec((BM, 128), lambda m: (m, 0)),
            pl.BlockSpec((din_a, exp), lambda m: (0, 0)),
            pl.BlockSpec((64, exp), lambda m: (0, 0)),
            pl.BlockSpec((1, exp), lambda m: (0, 0)),
            pl.BlockSpec((exp, gin), lambda m: (0, 0)),
            pl.BlockSpec((1, gin), lambda m: (0, 0)),
        ],
        out_specs=[
            pl.BlockSpec((BM, PW), lambda m: (m, 0)),
            pl.BlockSpec((BM, PW), lambda m: (m, 0)),
        ],
        out_shape=[
            jax.ShapeDtypeStruct((M, PW), F32),
            jax.ShapeDtypeStruct((M, PW), F32),
        ],
    )(nf_pad, emb_g, w1a, w1b, b1, w2, b2)


def _sage_mid_body(npan_in, *refs):
    apan = refs[:npan_in]
    xs = refs[npan_in:2 * npan_in]
    inv, lw, rw, lb = refs[2 * npan_in:2 * npan_in + 4]
    outs = refs[2 * npan_in + 4:]
    m = pl.program_id(0)
    a = jnp.concatenate([r[...] for r in apan], axis=1)
    mean = a * inv[...][:, 0:1]
    x = jnp.concatenate([r[...] for r in xs], axis=1)
    y = _bdot(mean, lw[...]) + _bdot(x, rw[...]) + lb[...]
    rows = m * BM + lax.broadcasted_iota(jnp.int32, (BM, 1), 0)
    maskf = (rows < N_NODES).astype(F32)
    y = jnp.where(maskf > 0, y, 0.0)
    nrm = jnp.sqrt(jnp.sum(y * y, axis=1, keepdims=True))
    z = y / jnp.maximum(nrm, 1e-12)
    z = jnp.where(z >= 0, z, 0.01 * z)
    for i, o in enumerate(outs):
        o[...] = z[:, i * PW:(i + 1) * PW]


def _run_sage_mid(apan, xs, inv128, lw, rw, lb, din):
    npan_in = len(apan)
    gh = lw.shape[1]
    npan_out = gh // PW
    body = functools.partial(_sage_mid_body, npan_in)
    pan_spec = pl.BlockSpec((BM, PW), lambda m: (m, 0))
    return pl.pallas_call(
        body,
        grid=(GRID_M,),
        in_specs=(
            [pan_spec] * (2 * npan_in)
            + [
                pl.BlockSpec((BM, 128), lambda m: (m, 0)),
                pl.BlockSpec((din, gh), lambda m: (0, 0)),
                pl.BlockSpec((din, gh), lambda m: (0, 0)),
                pl.BlockSpec((1, gh), lambda m: (0, 0)),
            ]
        ),
        out_specs=[pan_spec] * npan_out,
        out_shape=[jax.ShapeDtypeStruct((M, PW), F32)] * npan_out,
    )(*apan, *xs, inv128, lw, rw, lb)


def _sage_last_body(npan_in, *refs):
    apan = refs[:npan_in]
    xs = refs[npan_in:2 * npan_in]
    inv, lw, rw, lb, clsw, sep, prev = refs[2 * npan_in:2 * npan_in + 7]
    out = refs[2 * npan_in + 7]
    acc = refs[-1]
    m = pl.program_id(0)
    a = jnp.concatenate([r[...] for r in apan], axis=1)
    mean = a * inv[...][:, 0:1]
    x = jnp.concatenate([r[...] for r in xs], axis=1)
    y = _bdot(mean, lw[...]) + _bdot(x, rw[...]) + lb[...]
    rows = m * BM + lax.broadcasted_iota(jnp.int32, (BM, 1), 0)
    maskf = (rows < N_NODES).astype(F32)
    y = jnp.where(maskf > 0, y, 0.0)
    rowv = m * BM + lax.broadcasted_iota(jnp.int32, (16, BM), 1)
    s = sep[...][:, 0:1]
    p = prev[...][:, 0:1]
    ind = ((rowv >= p) & (rowv < s)).astype(F32)
    contrib = jnp.dot(ind, y, preferred_element_type=F32,
                      precision=lax.Precision.HIGHEST)

    @pl.when(m == 0)
    def _():
        acc[...] = jnp.zeros_like(acc)

    acc[...] += contrib

    @pl.when(m == GRID_M - 1)
    def _():
        out[...] = _bdot(acc[...], clsw[...])


def _run_sage_last(apan, xs, inv128, lw, rw, lb, cls_tiled, sep128, prev128,
                   din):
    npan_in = len(apan)
    gh = lw.shape[1]
    body = functools.partial(_sage_last_body, npan_in)
    pan_spec = pl.BlockSpec((BM, PW), lambda m: (m, 0))
    return pl.pallas_call(
        body,
        grid=(GRID_M,),
        in_specs=(
            [pan_spec] * (2 * npan_in)
            + [
                pl.BlockSpec((BM, 128), lambda m: (m, 0)),
                pl.BlockSpec((din, gh), lambda m: (0, 0)),
                pl.BlockSpec((din, gh), lambda m: (0, 0)),
                pl.BlockSpec((1, gh), lambda m: (0, 0)),
                pl.BlockSpec((gh, 128), lambda m: (0, 0)),
                pl.BlockSpec((16, 128), lambda m: (0, 0)),
                pl.BlockSpec((16, 128), lambda m: (0, 0)),
            ]
        ),
        out_specs=pl.BlockSpec((16, 128), lambda m: (0, 0)),
        out_shape=jax.ShapeDtypeStruct((16, 128), F32),
        scratch_shapes=[pltpu.VMEM((16, gh), F32)],
    )(*apan, *xs, inv128, lw, rw, lb, cls_tiled, sep128, prev128)


NC, NS = 2, 16
NW = NC * NS
K = 128
RA = 10112
FLUSH = RA // NS


def _sc_mesh():
    return plsc.VectorSubcoreMesh(core_axis_name="c", subcore_axis_name="s",
                                  num_cores=NC, num_subcores=NS)


@functools.partial(pl.kernel, mesh=_sc_mesh(),
                   out_type=jax.ShapeDtypeStruct((M, 128), F32),
                   scratch_types=[
                       pltpu.VMEM((2, K), jnp.int32),
                       pltpu.VMEM((2, K, 128), F32),
                       pltpu.SemaphoreType.DMA,
                   ])
def _embed_gather_sc(ops_hbm, emb_hbm, out_hbm, oidx, rows, sem):
    c = lax.axis_index("c")
    s = lax.axis_index("s")
    wid = s * NC + c
    nch = M // K
    nmine = (nch // NW) + jnp.where(wid < (nch % NW), 1, 0)

    def body(i, _):
        base = (wid + i * NW) * K
        pltpu.sync_copy(ops_hbm.at[pl.ds(base, K)], oidx.at[0])
        pltpu.async_copy(emb_hbm.at[oidx.at[0]], rows.at[0], sem).wait()
        pltpu.sync_copy(rows.at[0], out_hbm.at[pl.ds(base, K)])
        return 0

    lax.fori_loop(0, nmine, body, 0)


NCHT = 80
NCNT = 40
TRASH = 10100
EPAD = 1312 * K


@functools.partial(
    pl.kernel, mesh=_sc_mesh(),
    out_type=[jax.ShapeDtypeStruct((M, PW), F32) for _ in range(2)],
    scratch_types=[
        pltpu.VMEM((NCNT + 1, K), jnp.int32),
        pltpu.VMEM((K, PW), F32),
        pltpu.VMEM((K, PW), F32),
        pltpu.VMEM((K,), jnp.int32),
        pltpu.VMEM_SHARED((RA, PW), F32),
        pltpu.SemaphoreType.DMA,
    ])
def _count_sc(dst_hbm, ones_hbm, out0, out1, didx, onesbuf, zbuf, tidx,
              accum, sa):
    c = lax.axis_index("c")
    s = lax.axis_index("s")
    wid = s * NC + c
    outs = (out0, out1)

    def wait_full():
        pltpu.make_async_copy(ones_hbm, onesbuf, sa).wait()

    def wait_idx():
        pltpu.make_async_copy(dst_hbm.at[pl.ds(0, K)], tidx, sa).wait()

    def pre(i, _):
        @pl.when(i >= 8)
        def _():
            wait_idx()

        pltpu.async_copy(dst_hbm.at[pl.ds((wid + i * NW) * K, K)], didx.at[i],
                         sa)
        return 0

    lax.fori_loop(0, NCNT, pre, 0)
    pltpu.sync_copy(ones_hbm, onesbuf)
    zbuf[...] = jnp.zeros_like(zbuf)
    for _ in range(8):
        wait_idx()

    for r in range(FLUSH // K):
        pltpu.async_copy(zbuf, accum.at[pl.ds(s * FLUSH + r * K, K)], sa)
    rem = FLUSH % K
    if rem:
        pltpu.async_copy(zbuf.at[pl.ds(0, rem)],
                         accum.at[pl.ds(s * FLUSH + (FLUSH // K) * K, rem)],
                         sa)
    for r in range(FLUSH // K):
        wait_full()
    if rem:
        pltpu.make_async_copy(ones_hbm.at[pl.ds(0, rem)],
                              onesbuf.at[pl.ds(0, rem)], sa).wait()
    plsc.subcore_barrier()

    def body(i, _):
        @pl.when(i >= 4)
        def _():
            wait_full()

        pltpu.async_copy(onesbuf, accum.at[didx.at[i]], sa, add=True)
        return 0

    lax.fori_loop(0, NCNT, body, 0)
    for _ in range(4):
        wait_full()
    plsc.subcore_barrier()

    for ci in range(NC):
        @pl.when(c == ci)
        def _(ci=ci):
            pltpu.sync_copy(accum.at[pl.ds(s * FLUSH, FLUSH)],
                            outs[ci].at[pl.ds(s * FLUSH, FLUSH)])

        @pl.when((c == ci) & (s == NS - 1))
        def _(ci=ci):
            pltpu.sync_copy(zbuf, outs[ci].at[pl.ds(RA, M - RA)])


ZB = 8


@functools.partial(
    pl.kernel, mesh=_sc_mesh(),
    out_type=[jax.ShapeDtypeStruct((M, PW), F32) for _ in range(2)],
    scratch_types=[
        pltpu.VMEM((NCHT + 1, K), jnp.int32),
        pltpu.VMEM((4, K), jnp.int32),
        pltpu.VMEM((2, K, PW), F32),
        pltpu.VMEM((ZB, PW), F32),
        pltpu.VMEM((K,), jnp.int32),
        pltpu.VMEM_SHARED((RA, PW), F32),
        pltpu.SemaphoreType.DMA,
        pltpu.SemaphoreType.DMA,
        pltpu.SemaphoreType.DMA,
        pltpu.SemaphoreType.DMA,
        pltpu.SemaphoreType.DMA,
    ])
def _agg_pair(src_hbm, dst_hbm, zeros_hbm, xp0, xp1, out0, out1,
              sidx, didx, rows, zbuf, tidx, accum, semi, g0, g1, s0, s1):
    c = lax.axis_index("c")
    s = lax.axis_index("s")
    xps = (xp0, xp1)
    outs = (out0, out1)

    def wait_rows(sem):
        pltpu.make_async_copy(xp0.at[pl.ds(0, K)], rows.at[0], sem).wait()

    def wait_z(sem):
        pltpu.make_async_copy(xp0.at[pl.ds(0, ZB)], rows.at[0, pl.ds(0, ZB)],
                              sem).wait()

    def wait_idx(sem):
        pltpu.make_async_copy(src_hbm.at[pl.ds(0, K)], tidx, sem).wait()

    def didx_load(i):
        pltpu.async_copy(dst_hbm.at[pl.ds((s + i * NS) * K, K)],
                         didx.at[lax.rem(i, 4)], semi)

    def pre(i, _):
        @pl.when(i >= 8)
        def _():
            wait_idx(semi)

        pltpu.async_copy(src_hbm.at[pl.ds((s + i * NS) * K, K)], sidx.at[i],
                         semi)
        return 0

    lax.fori_loop(0, NCHT + 1, pre, 0)
    pltpu.sync_copy(zeros_hbm, zbuf)
    pltpu.sync_copy(dst_hbm.at[pl.ds(EPAD - K, K)], tidx)
    for _ in range(8):
        wait_idx(semi)
    didx_load(0)
    didx_load(1)

    for ci in range(NC):
        @pl.when(c == ci)
        def _(ci=ci):
            pltpu.async_copy(xps[ci].at[sidx.at[0]], rows.at[0], g0)

    def zbody(r, _):
        @pl.when(r >= 8)
        def _():
            wait_z(s0)

        pltpu.async_copy(zbuf, accum.at[pl.ds(s * FLUSH + r * ZB, ZB)], s0)
        return 0

    lax.fori_loop(0, FLUSH // ZB, zbody, 0)
    for _ in range(8):
        wait_z(s0)
    plsc.subcore_barrier()
    pltpu.async_copy(rows.at[1], accum.at[tidx], s1, add=True)

    for ci in range(NC):
        @pl.when(c == ci)
        def _(ci=ci):
            def body(h, _):
                i0 = 2 * h
                wait_rows(s1)
                wait_idx(semi)
                wait_idx(semi)
                didx_load(i0 + 2)
                didx_load(i0 + 3)
                pltpu.async_copy(xps[ci].at[sidx.at[i0 + 1]], rows.at[1], g1)
                wait_rows(g0)
                pltpu.async_copy(rows.at[0], accum.at[didx.at[lax.rem(i0, 4)]],
                                 s0, add=True)
                wait_rows(s0)
                pltpu.async_copy(xps[ci].at[sidx.at[i0 + 2]], rows.at[0], g0)
                wait_rows(g1)
                pltpu.async_copy(rows.at[1],
                                 accum.at[didx.at[lax.rem(i0 + 1, 4)]], s1,
                                 add=True)
                return 0

            lax.fori_loop(0, NCHT // 2, body, 0)

    wait_rows(g0)
    wait_rows(s1)
    wait_idx(semi)
    wait_idx(semi)
    plsc.subcore_barrier()

    for ci in range(NC):
        @pl.when(c == ci)
        def _(ci=ci):
            pltpu.sync_copy(accum.at[pl.ds(s * FLUSH, FLUSH)],
                            outs[ci].at[pl.ds(s * FLUSH, FLUSH)])

        @pl.when((c == ci) & (s == NS - 1))
        def _(ci=ci):
            for r in range((M - RA) // ZB):
                pltpu.sync_copy(zbuf, outs[ci].at[pl.ds(RA + r * ZB, ZB)])


def _aggregate(x_panels, src, dst):
    zeros = jnp.zeros((ZB, PW), F32)
    outs = []
    for p in range(0, len(x_panels), 2):
        outs.extend(_agg_pair(src, dst, zeros, x_panels[p], x_panels[p + 1]))
    return outs


def kernel(node_features, node_separation, node_ops, edges, batches, opcode_emb,
           mlp_W1, mlp_b1, mlp_W2, mlp_b2,
           s0_lW, s0_lb, s0_rW, s1_lW, s1_lb, s1_rW, s2_lW, s2_lb, s2_rW,
           cls_W, cls_b):
    pad = M - N_NODES
    nf_pad = jnp.pad(node_features, ((0, pad), (0, 0)))
    ops_pad = jnp.pad(node_ops, (0, pad))
    epad = EPAD - E_EDGES
    src = jnp.pad(edges[0], (0, epad))
    dst = jnp.pad(edges[1], (0, epad), constant_values=TRASH)

    emb128 = jnp.pad(opcode_emb, ((0, 0), (0, 128 - 64)))
    emb_g = _embed_gather_sc(ops_pad, emb128)

    ones1 = jnp.zeros((K, PW), F32).at[:, 0].set(1.0)
    cntA, cntB = _count_sc(dst, ones1)
    cnt = cntA[:, 0:1] + cntB[:, 0:1]
    inv128 = jnp.broadcast_to(1.0 / jnp.maximum(cnt, 1.0), (M, 128))

    w1a, w1b = mlp_W1[:126], mlp_W1[126:]
    x0p = _run_mlp(nf_pad, emb_g, w1a, w1b, mlp_b1.reshape(1, -1),
                   mlp_W2, mlp_b2.reshape(1, -1))

    a0 = _aggregate(x0p, src, dst)
    x1p = _run_sage_mid(a0, x0p, inv128, s0_lW, s0_rW,
                        s0_lb.reshape(1, -1), din=256)
    a1 = _aggregate(x1p, src, dst)
    x2p = _run_sage_mid(a1, x1p, inv128, s1_lW, s1_rW,
                        s1_lb.reshape(1, -1), din=512)
    a2 = _aggregate(x2p, src, dst)
    sep128 = jnp.broadcast_to(node_separation.reshape(16, 1), (16, 128))
    prev = jnp.concatenate([jnp.zeros((1,), node_separation.dtype),
                            node_separation[:15]])
    prev128 = jnp.broadcast_to(prev.reshape(16, 1), (16, 128))
    cls_tiled = jnp.broadcast_to(cls_W, (cls_W.shape[0], 128))
    out128 = _run_sage_last(a2, x2p, inv128, s2_lW, s2_rW,
                            s2_lb.reshape(1, -1), cls_tiled, sep128, prev128,
                            din=512)

    t = out128[:, 0]
    return jnp.zeros((16,), F32).at[batches].set(t) + cls_b
